# initial kernel scaffold (unmeasured)
import jax
import jax.numpy as jnp
from jax import lax
from jax.experimental import pallas as pl
from jax.experimental.pallas import tpu as pltpu

N_DEV = 4
N_TILE = 2048


def kernel(x, w_mat, scale_x, scale_w):
    m_total, k_loc = x.shape
    _, n = w_mat.shape
    m_chunk = m_total // N_DEV
    n_tiles = n // N_TILE

    def body(x_ref, w_ref, sx_ref, sw_ref, out_ref,
             send_buf, recv_buf, send_sems, recv_sems, credit_sem):
        my = lax.axis_index("i")
        left = lax.rem(my - 1 + N_DEV, N_DEV)
        right = lax.rem(my + 1, N_DEV)

        barrier = pltpu.get_barrier_semaphore()
        for nbr in (left, right):
            pl.semaphore_signal(barrier, inc=1, device_id=(nbr,),
                                device_id_type=pl.DeviceIdType.MESH)
        pl.semaphore_wait(barrier, 2)

        s_scale = sx_ref[0] * sw_ref[0]

        def partial_tile(c, t):
            xt = x_ref[pl.ds(c * m_chunk, m_chunk), :].astype(jnp.bfloat16)
            wt = w_ref[:, t * N_TILE:(t + 1) * N_TILE].astype(jnp.bfloat16)
            return lax.dot_general(xt, wt, (((1,), (0,)), ((), ())),
                                   preferred_element_type=jnp.float32)

        for s in range(N_DEV - 1):
            c = lax.rem(my - 1 - s + 2 * N_DEV, N_DEV)
            for t in range(n_tiles):
                acc = partial_tile(c, t)
                if s > 0:
                    acc = acc + recv_buf[
                        (s - 1) % 2, :, t * N_TILE:(t + 1) * N_TILE
                    ].astype(jnp.float32)
                send_buf[:, t * N_TILE:(t + 1) * N_TILE] = acc.astype(jnp.bfloat16)

            if s == 1:
                pl.semaphore_signal(credit_sem, inc=1, device_id=(left,),
                                    device_id_type=pl.DeviceIdType.MESH)
            if s == 2:
                pl.semaphore_wait(credit_sem, 1)

            rdma = pltpu.make_async_remote_copy(
                src_ref=send_buf,
                dst_ref=recv_buf.at[s % 2],
                send_sem=send_sems.at[s],
                recv_sem=recv_sems.at[s],
                device_id=(right,),
                device_id_type=pl.DeviceIdType.MESH,
            )
            rdma.start()
            rdma.wait()

        for t in range(n_tiles):
            acc = partial_tile(my, t) + recv_buf[
                0, :, t * N_TILE:(t + 1) * N_TILE
            ].astype(jnp.float32)
            y = acc * s_scale
            out_ref[:, t * N_TILE:(t + 1) * N_TILE] = y * jax.nn.sigmoid(y)

    return pl.pallas_call(
        body,
        out_shape=jax.ShapeDtypeStruct((m_chunk, n), jnp.float32),
        in_specs=[
            pl.BlockSpec(memory_space=pltpu.VMEM),
            pl.BlockSpec(memory_space=pltpu.VMEM),
            pl.BlockSpec(memory_space=pltpu.SMEM),
            pl.BlockSpec(memory_space=pltpu.SMEM),
        ],
        out_specs=pl.BlockSpec(memory_space=pltpu.VMEM),
        scratch_shapes=[
            pltpu.VMEM((m_chunk, n), jnp.bfloat16),
            pltpu.VMEM((2, m_chunk, n), jnp.bfloat16),
            pltpu.SemaphoreType.DMA((N_DEV - 1,)),
            pltpu.SemaphoreType.DMA((N_DEV - 1,)),
            pltpu.SemaphoreType.REGULAR,
        ],
        compiler_params=pltpu.CompilerParams(collective_id=0),
    )(x, w_mat, scale_x, scale_w)


# baseline (device time: 735408 ns/iter reference)
import jax
import jax.numpy as jnp
from jax import lax
from jax.experimental import pallas as pl
from jax.experimental.pallas import tpu as pltpu

N_DEV = 4
NB = 1024
N_BLOCKS = 8192 // NB


def kernel(x, w_mat, scale_x, scale_w):
    m_total, k_loc = x.shape
    _, n = w_mat.shape
    m_chunk = m_total // N_DEV
    n_blocks = n // NB

    x8 = x.astype(jnp.float8_e4m3fn)
    w8 = w_mat.astype(jnp.float8_e5m2)

    def body(x_ref, w_ref, sx_ref, sw_ref, out_ref,
             send_buf, recv_buf, out_stage,
             send_sems, recv_sems, copy_sem, credit_sem):
        my = lax.axis_index("i")
        left = lax.rem(my - 1 + N_DEV, N_DEV)
        right = lax.rem(my + 1, N_DEV)

        barrier = pltpu.get_barrier_semaphore()
        for nbr in (left, right):
            pl.semaphore_signal(barrier, inc=1, device_id=(nbr,),
                                device_id_type=pl.DeviceIdType.MESH)
        pl.semaphore_wait(barrier, 2)

        s_scale = sx_ref[0] * sw_ref[0]

        def partial_tile(c, b):
            xt = x_ref[pl.ds(c * m_chunk, m_chunk), :].astype(jnp.bfloat16)
            wt = w_ref[:, b * NB:(b + 1) * NB].astype(jnp.bfloat16)
            return lax.dot_general(xt, wt, (((1,), (0,)), ((), ())),
                                   preferred_element_type=jnp.float32)

        for s in range(N_DEV - 1):
            c = lax.rem(my - 1 - s + 2 * N_DEV, N_DEV)
            for b in range(n_blocks):
                acc = partial_tile(c, b)
                if s > 0:
                    acc = acc + recv_buf[b].astype(jnp.float32)
                send_buf[b % 2] = acc.astype(jnp.bfloat16)

                if s > 0:
                    pl.semaphore_signal(credit_sem, inc=1, device_id=(left,),
                                        device_id_type=pl.DeviceIdType.MESH)
                    pl.semaphore_wait(credit_sem, 1)

                rdma = pltpu.make_async_remote_copy(
                    src_ref=send_buf.at[b % 2],
                    dst_ref=recv_buf.at[b],
                    send_sem=send_sems.at[b % 2],
                    recv_sem=recv_sems.at[b],
                    device_id=(right,),
                    device_id_type=pl.DeviceIdType.MESH,
                )
                rdma.start()
                rdma.wait()

        for b in range(n_blocks):
            acc = partial_tile(my, b) + recv_buf[b].astype(jnp.float32)
            y = acc * s_scale
            out_stage[0] = y * jax.nn.sigmoid(y)
            cp = pltpu.make_async_copy(
                out_stage.at[0], out_ref.at[:, pl.ds(b * NB, NB)], copy_sem)
            cp.start()
            cp.wait()

    return pl.pallas_call(
        body,
        out_shape=jax.ShapeDtypeStruct((m_chunk, n), jnp.float32),
        in_specs=[
            pl.BlockSpec(memory_space=pltpu.VMEM),
            pl.BlockSpec(memory_space=pltpu.VMEM),
            pl.BlockSpec(memory_space=pltpu.SMEM),
            pl.BlockSpec(memory_space=pltpu.SMEM),
        ],
        out_specs=pl.BlockSpec(memory_space=pl.ANY),
        scratch_shapes=[
            pltpu.VMEM((2, m_chunk, NB), jnp.bfloat16),
            pltpu.VMEM((N_BLOCKS, m_chunk, NB), jnp.bfloat16),
            pltpu.VMEM((1, m_chunk, NB), jnp.float32),
            pltpu.SemaphoreType.DMA((2,)),
            pltpu.SemaphoreType.DMA((N_BLOCKS,)),
            pltpu.SemaphoreType.DMA,
            pltpu.SemaphoreType.REGULAR,
        ],
        compiler_params=pltpu.CompilerParams(collective_id=0),
    )(x8, w8, scale_x, scale_w)


# device time: 408444 ns/iter; 1.8005x vs baseline; 1.8005x over previous
import jax
import jax.numpy as jnp
from jax import lax
from jax.experimental import pallas as pl
from jax.experimental.pallas import tpu as pltpu

N_DEV = 4
NB = 1024
N_BLOCKS = 8192 // NB
NBC = NB // 2


def kernel(x, w_mat, scale_x, scale_w):
    m_total, k_loc = x.shape
    _, n = w_mat.shape
    m_chunk = m_total // N_DEV
    n_blocks = n // NB
    half = n_blocks // 2
    n_msgs = (N_DEV - 1) * n_blocks

    x8 = x.astype(jnp.float8_e4m3fn)
    w8 = w_mat.astype(jnp.float8_e5m2)

    def body(x_ref, w_ref, sx_ref, sw_ref, out_ref,
             send_buf, recv_buf, out_stage,
             send_sems, recv_sems, copy_sems, credit_cw, credit_ccw):
        my = lax.axis_index("i")
        left = lax.rem(my - 1 + N_DEV, N_DEV)
        right = lax.rem(my + 1, N_DEV)

        barrier = pltpu.get_barrier_semaphore()
        for nbr in (left, right):
            pl.semaphore_signal(barrier, inc=1, device_id=(nbr,),
                                device_id_type=pl.DeviceIdType.MESH)
        pl.semaphore_wait(barrier, 2)

        s_scale = sx_ref[0] * sw_ref[0]

        def partial_tile(c, b, h):
            xt = x_ref[pl.ds(c * m_chunk, m_chunk), :].astype(jnp.bfloat16)
            wt = w_ref[:, pl.ds(b * NB + h * NBC, NBC)].astype(jnp.bfloat16)
            return lax.dot_general(xt, wt, (((1,), (0,)), ((), ())),
                                   preferred_element_type=jnp.float32)

        def send_rdma(slot, b, dest):
            return pltpu.make_async_remote_copy(
                src_ref=send_buf.at[slot],
                dst_ref=recv_buf.at[b],
                send_sem=send_sems.at[slot],
                recv_sem=recv_sems.at[b],
                device_id=(dest,),
                device_id_type=pl.DeviceIdType.MESH,
            )

        def wait_recv(b):
            send_rdma(0, b, my).wait_recv()

        def wait_send(slot):
            send_rdma(slot, 0, my).wait_send()

        def ring_msg(k, carry):
            s = lax.div(k, n_blocks)
            b = lax.rem(k, n_blocks)
            cw = b < half
            dest = jnp.where(cw, right, left)
            upstream = jnp.where(cw, left, right)
            slot = lax.rem(b, 2) + jnp.where(cw, 0, 2)
            c = jnp.where(
                cw,
                lax.rem(my - 1 - s + 2 * N_DEV, N_DEV),
                lax.rem(my + 1 + s, N_DEV),
            )

            @pl.when(s > 0)
            def _():
                wait_recv(b)

            @pl.when((s > 0) | (lax.rem(b, half) >= 2))
            def _():
                wait_send(slot)

            for h in range(2):
                acc = partial_tile(c, b, h)
                acc = jnp.where(
                    s > 0,
                    acc + recv_buf[b, :, h * NBC:(h + 1) * NBC].astype(
                        jnp.float32),
                    acc,
                )
                send_buf[slot, :, h * NBC:(h + 1) * NBC] = (
                    acc.astype(jnp.bfloat16))

            @pl.when((s > 0) & cw)
            def _():
                pl.semaphore_signal(credit_cw, inc=1, device_id=(left,),
                                    device_id_type=pl.DeviceIdType.MESH)
                pl.semaphore_wait(credit_cw, 1)

            @pl.when((s > 0) & jnp.logical_not(cw))
            def _():
                pl.semaphore_signal(credit_ccw, inc=1, device_id=(right,),
                                    device_id_type=pl.DeviceIdType.MESH)
                pl.semaphore_wait(credit_ccw, 1)

            send_rdma(slot, b, dest).start()
            return carry

        lax.fori_loop(0, n_msgs, ring_msg, 0)

        def final_msg(b, carry):
            wait_recv(b)
            st = lax.rem(b, 2)

            @pl.when(b >= 2)
            def _():
                pltpu.make_async_copy(
                    out_stage.at[st],
                    out_ref.at[:, pl.ds((b - 2) * NB, NB)],
                    copy_sems.at[st]).wait()

            for h in range(2):
                acc = partial_tile(my, b, h) + recv_buf[
                    b, :, h * NBC:(h + 1) * NBC].astype(jnp.float32)
                y = acc * s_scale
                out_stage[st, :, h * NBC:(h + 1) * NBC] = (
                    y * jax.nn.sigmoid(y))

            pltpu.make_async_copy(
                out_stage.at[st], out_ref.at[:, pl.ds(b * NB, NB)],
                copy_sems.at[st]).start()
            return carry

        lax.fori_loop(0, n_blocks, final_msg, 0)

        for b in range(n_blocks - 2, n_blocks):
            st = b % 2
            pltpu.make_async_copy(
                out_stage.at[st], out_ref.at[:, pl.ds(b * NB, NB)],
                copy_sems.at[st]).wait()
        for slot in range(4):
            wait_send(slot)

    return pl.pallas_call(
        body,
        out_shape=jax.ShapeDtypeStruct((m_chunk, n), jnp.float32),
        in_specs=[
            pl.BlockSpec(memory_space=pltpu.VMEM),
            pl.BlockSpec(memory_space=pltpu.VMEM),
            pl.BlockSpec(memory_space=pltpu.SMEM),
            pl.BlockSpec(memory_space=pltpu.SMEM),
        ],
        out_specs=pl.BlockSpec(memory_space=pl.ANY),
        scratch_shapes=[
            pltpu.VMEM((4, m_chunk, NB), jnp.bfloat16),
            pltpu.VMEM((N_BLOCKS, m_chunk, NB), jnp.bfloat16),
            pltpu.VMEM((2, m_chunk, NB), jnp.float32),
            pltpu.SemaphoreType.DMA((4,)),
            pltpu.SemaphoreType.DMA((N_BLOCKS,)),
            pltpu.SemaphoreType.DMA((2,)),
            pltpu.SemaphoreType.REGULAR,
            pltpu.SemaphoreType.REGULAR,
        ],
        compiler_params=pltpu.CompilerParams(
            collective_id=0, vmem_limit_bytes=48 * 1024 * 1024),
    )(x8, w8, scale_x, scale_w)


# device time: 338022 ns/iter; 2.1756x vs baseline; 1.2083x over previous
import jax
import jax.numpy as jnp
from jax import lax
from jax.experimental import pallas as pl
from jax.experimental.pallas import tpu as pltpu

N_DEV = 4
NB = 1024
N_BLOCKS = 8192 // NB
NBC = NB // 2


def kernel(x, w_mat, scale_x, scale_w):
    m_total, k_loc = x.shape
    _, n = w_mat.shape
    m_chunk = m_total // N_DEV
    n_blocks = n // NB
    half = n_blocks // 2
    n_msgs = (N_DEV - 1) * n_blocks

    x8 = x.astype(jnp.bfloat16)
    w8 = w_mat.astype(jnp.float8_e5m2)

    def body(x_ref, w_ref, sx_ref, sw_ref, out_ref,
             send_buf, recv_buf, out_stage,
             send_sems, recv_sems, copy_sems, credit_cw, credit_ccw):
        my = lax.axis_index("i")
        left = lax.rem(my - 1 + N_DEV, N_DEV)
        right = lax.rem(my + 1, N_DEV)

        barrier = pltpu.get_barrier_semaphore()
        for nbr in (left, right):
            pl.semaphore_signal(barrier, inc=1, device_id=(nbr,),
                                device_id_type=pl.DeviceIdType.MESH)
        pl.semaphore_wait(barrier, 2)

        s_scale = sx_ref[0] * sw_ref[0]

        def partial_tile(c, b, h):
            xt = x_ref[pl.ds(c * m_chunk, m_chunk), :]
            wt = w_ref[:, pl.ds(b * NB + h * NBC, NBC)].astype(jnp.bfloat16)
            return lax.dot_general(xt, wt, (((1,), (0,)), ((), ())),
                                   preferred_element_type=jnp.float32)

        def send_rdma(slot, b, dest):
            return pltpu.make_async_remote_copy(
                src_ref=send_buf.at[slot],
                dst_ref=recv_buf.at[b],
                send_sem=send_sems.at[slot],
                recv_sem=recv_sems.at[b],
                device_id=(dest,),
                device_id_type=pl.DeviceIdType.MESH,
            )

        def wait_recv(b):
            send_rdma(0, b, my).wait_recv()

        def wait_send(slot):
            send_rdma(slot, 0, my).wait_send()

        def ring_msg(k, carry):
            s = lax.div(k, n_blocks)
            j = lax.rem(k, n_blocks)
            b = lax.rem(j, 2) * half + lax.div(j, 2)
            cw = b < half
            dest = jnp.where(cw, right, left)
            upstream = jnp.where(cw, left, right)
            slot = lax.rem(b, 2) + jnp.where(cw, 0, 2)
            c = jnp.where(
                cw,
                lax.rem(my - 1 - s + 2 * N_DEV, N_DEV),
                lax.rem(my + 1 + s, N_DEV),
            )

            @pl.when(s > 0)
            def _():
                wait_recv(b)

            @pl.when((s > 0) | (lax.rem(b, half) >= 2))
            def _():
                wait_send(slot)

            for h in range(2):
                acc = partial_tile(c, b, h)
                acc = jnp.where(
                    s > 0,
                    acc + recv_buf[b, :, h * NBC:(h + 1) * NBC].astype(
                        jnp.float32),
                    acc,
                )
                send_buf[slot, :, h * NBC:(h + 1) * NBC] = (
                    acc.astype(jnp.bfloat16))

            @pl.when((s > 0) & cw)
            def _():
                pl.semaphore_signal(credit_cw, inc=1, device_id=(left,),
                                    device_id_type=pl.DeviceIdType.MESH)
                pl.semaphore_wait(credit_cw, 1)

            @pl.when((s > 0) & jnp.logical_not(cw))
            def _():
                pl.semaphore_signal(credit_ccw, inc=1, device_id=(right,),
                                    device_id_type=pl.DeviceIdType.MESH)
                pl.semaphore_wait(credit_ccw, 1)

            send_rdma(slot, b, dest).start()
            return carry

        lax.fori_loop(0, n_msgs, ring_msg, 0)

        def final_msg(j, carry):
            b = lax.rem(j, 2) * half + lax.div(j, 2)
            wait_recv(b)
            st = lax.rem(j, 2)

            @pl.when(j >= 2)
            def _():
                pltpu.make_async_copy(
                    out_stage.at[st],
                    out_ref.at[:, pl.ds((b - 1) * NB, NB)],
                    copy_sems.at[st]).wait()

            for h in range(2):
                acc = partial_tile(my, b, h) + recv_buf[
                    b, :, h * NBC:(h + 1) * NBC].astype(jnp.float32)
                y = acc * s_scale
                out_stage[st, :, h * NBC:(h + 1) * NBC] = (
                    y * jax.nn.sigmoid(y))

            pltpu.make_async_copy(
                out_stage.at[st], out_ref.at[:, pl.ds(b * NB, NB)],
                copy_sems.at[st]).start()
            return carry

        lax.fori_loop(0, n_blocks, final_msg, 0)

        for st, b in ((0, half - 1), (1, n_blocks - 1)):
            pltpu.make_async_copy(
                out_stage.at[st], out_ref.at[:, pl.ds(b * NB, NB)],
                copy_sems.at[st]).wait()
        for slot in range(4):
            wait_send(slot)

    return pl.pallas_call(
        body,
        out_shape=jax.ShapeDtypeStruct((m_chunk, n), jnp.float32),
        in_specs=[
            pl.BlockSpec(memory_space=pltpu.VMEM),
            pl.BlockSpec(memory_space=pltpu.VMEM),
            pl.BlockSpec(memory_space=pltpu.SMEM),
            pl.BlockSpec(memory_space=pltpu.SMEM),
        ],
        out_specs=pl.BlockSpec(memory_space=pl.ANY),
        scratch_shapes=[
            pltpu.VMEM((4, m_chunk, NB), jnp.bfloat16),
            pltpu.VMEM((N_BLOCKS, m_chunk, NB), jnp.bfloat16),
            pltpu.VMEM((2, m_chunk, NB), jnp.float32),
            pltpu.SemaphoreType.DMA((4,)),
            pltpu.SemaphoreType.DMA((N_BLOCKS,)),
            pltpu.SemaphoreType.DMA((2,)),
            pltpu.SemaphoreType.REGULAR,
            pltpu.SemaphoreType.REGULAR,
        ],
        compiler_params=pltpu.CompilerParams(
            collective_id=0, vmem_limit_bytes=58 * 1024 * 1024),
    )(x8, w8, scale_x, scale_w)


# device time: 333262 ns/iter; 2.2067x vs baseline; 1.0143x over previous
import jax
import jax.numpy as jnp
from jax import lax
from jax.experimental import pallas as pl
from jax.experimental.pallas import tpu as pltpu

N_DEV = 4
NB = 1024
N_BLOCKS = 8192 // NB
NBC = NB // 2


def kernel(x, w_mat, scale_x, scale_w):
    m_total, k_loc = x.shape
    _, n = w_mat.shape
    m_chunk = m_total // N_DEV
    n_blocks = n // NB
    half = n_blocks // 2
    n_msgs = (N_DEV - 1) * n_blocks

    x8 = x.astype(jnp.float8_e4m3fn)
    w8 = w_mat.astype(jnp.float8_e5m2)

    def body(x_ref, w_ref, sx_ref, sw_ref, out_ref,
             send_buf, recv_buf, out_stage,
             send_sems, recv_sems, copy_sems, credit_cw, credit_ccw):
        my = lax.axis_index("i")
        left = lax.rem(my - 1 + N_DEV, N_DEV)
        right = lax.rem(my + 1, N_DEV)

        barrier = pltpu.get_barrier_semaphore()
        for nbr in (left, right):
            pl.semaphore_signal(barrier, inc=1, device_id=(nbr,),
                                device_id_type=pl.DeviceIdType.MESH)
        pl.semaphore_wait(barrier, 2)

        s_scale = sx_ref[0] * sw_ref[0]

        def partial_tile(c, b, h):
            xt = x_ref[pl.ds(c * m_chunk, m_chunk), :]
            wt = w_ref[:, pl.ds(b * NB + h * NBC, NBC)]
            return lax.dot_general(xt, wt, (((1,), (0,)), ((), ())),
                                   preferred_element_type=jnp.float32)

        def send_rdma(slot, b, dest):
            return pltpu.make_async_remote_copy(
                src_ref=send_buf.at[slot],
                dst_ref=recv_buf.at[b],
                send_sem=send_sems.at[slot],
                recv_sem=recv_sems.at[b],
                device_id=(dest,),
                device_id_type=pl.DeviceIdType.MESH,
            )

        def wait_recv(b):
            send_rdma(0, b, my).wait_recv()

        def wait_send(slot):
            send_rdma(slot, 0, my).wait_send()

        def ring_msg(k, carry):
            s = lax.div(k, n_blocks)
            j = lax.rem(k, n_blocks)
            b = lax.rem(j, 2) * half + lax.div(j, 2)
            cw = b < half
            dest = jnp.where(cw, right, left)
            upstream = jnp.where(cw, left, right)
            slot = lax.rem(b, 2) + jnp.where(cw, 0, 2)
            c = jnp.where(
                cw,
                lax.rem(my - 1 - s + 2 * N_DEV, N_DEV),
                lax.rem(my + 1 + s, N_DEV),
            )

            @pl.when(s > 0)
            def _():
                wait_recv(b)

            @pl.when((s > 0) | (lax.rem(b, half) >= 2))
            def _():
                wait_send(slot)

            for h in range(2):
                acc = partial_tile(c, b, h)
                acc = jnp.where(
                    s > 0,
                    acc + recv_buf[b, :, h * NBC:(h + 1) * NBC].astype(
                        jnp.float32),
                    acc,
                )
                send_buf[slot, :, h * NBC:(h + 1) * NBC] = (
                    acc.astype(jnp.bfloat16))

            @pl.when((s > 0) & cw)
            def _():
                pl.semaphore_signal(credit_cw, inc=1, device_id=(left,),
                                    device_id_type=pl.DeviceIdType.MESH)
                pl.semaphore_wait(credit_cw, 1)

            @pl.when((s > 0) & jnp.logical_not(cw))
            def _():
                pl.semaphore_signal(credit_ccw, inc=1, device_id=(right,),
                                    device_id_type=pl.DeviceIdType.MESH)
                pl.semaphore_wait(credit_ccw, 1)

            send_rdma(slot, b, dest).start()
            return carry

        lax.fori_loop(0, n_msgs, ring_msg, 0)

        def final_msg(j, carry):
            b = lax.rem(j, 2) * half + lax.div(j, 2)
            wait_recv(b)
            st = lax.rem(j, 2)

            @pl.when(j >= 2)
            def _():
                pltpu.make_async_copy(
                    out_stage.at[st],
                    out_ref.at[:, pl.ds((b - 1) * NB, NB)],
                    copy_sems.at[st]).wait()

            for h in range(2):
                acc = partial_tile(my, b, h) + recv_buf[
                    b, :, h * NBC:(h + 1) * NBC].astype(jnp.float32)
                y = acc * s_scale
                out_stage[st, :, h * NBC:(h + 1) * NBC] = (
                    y * jax.nn.sigmoid(y))

            pltpu.make_async_copy(
                out_stage.at[st], out_ref.at[:, pl.ds(b * NB, NB)],
                copy_sems.at[st]).start()
            return carry

        lax.fori_loop(0, n_blocks, final_msg, 0)

        for st, b in ((0, half - 1), (1, n_blocks - 1)):
            pltpu.make_async_copy(
                out_stage.at[st], out_ref.at[:, pl.ds(b * NB, NB)],
                copy_sems.at[st]).wait()
        for slot in range(4):
            wait_send(slot)

    return pl.pallas_call(
        body,
        out_shape=jax.ShapeDtypeStruct((m_chunk, n), jnp.float32),
        in_specs=[
            pl.BlockSpec(memory_space=pltpu.VMEM),
            pl.BlockSpec(memory_space=pltpu.VMEM),
            pl.BlockSpec(memory_space=pltpu.SMEM),
            pl.BlockSpec(memory_space=pltpu.SMEM),
        ],
        out_specs=pl.BlockSpec(memory_space=pl.ANY),
        scratch_shapes=[
            pltpu.VMEM((4, m_chunk, NB), jnp.bfloat16),
            pltpu.VMEM((N_BLOCKS, m_chunk, NB), jnp.bfloat16),
            pltpu.VMEM((2, m_chunk, NB), jnp.float32),
            pltpu.SemaphoreType.DMA((4,)),
            pltpu.SemaphoreType.DMA((N_BLOCKS,)),
            pltpu.SemaphoreType.DMA((2,)),
            pltpu.SemaphoreType.REGULAR,
            pltpu.SemaphoreType.REGULAR,
        ],
        compiler_params=pltpu.CompilerParams(
            collective_id=0, vmem_limit_bytes=58 * 1024 * 1024),
    )(x8, w8, scale_x, scale_w)
